# Initial kernel scaffold; baseline (speedup 1.0000x reference)
import jax
import jax.numpy as jnp
from jax.experimental import pallas as pl


def kernel(x, edge_index, batch, d0W, d0b, d1W, d1b, d2W, d2b, d3W, d3b, d4W, d4b, p0, p1, p2, p3, u0W, u0b, u1W, u1b, u2W, u2b, u3W, u3b, c1W, c1b, c2W, c2b, oW, ob):
    raise NotImplementedError("write your pallas kernel here")


# speedup vs baseline: 1.6679x; 1.6679x over previous
import functools
import math

import jax
import jax.numpy as jnp
from jax.experimental import pallas as pl
from jax.experimental.pallas import tpu as pltpu


def _pick_block(n):
    for b in (512, 256):
        if n % b == 0:
            return b
    return n


def _pool_mm_kernel(r_ref, c_ref, o_ref, ot_ref, acc_ref, *, nk, bm, bn, k_real):
    @pl.when(pl.program_id(2) == 0)
    def _init():
        acc_ref[...] = jnp.zeros_like(acc_ref)

    acc_ref[...] += jax.lax.dot_general(
        r_ref[...], c_ref[...], (((1,), (1,)), ((), ())),
        preferred_element_type=jnp.float32)

    @pl.when(pl.program_id(2) == nk - 1)
    def _fin():
        i = pl.program_id(0)
        j = pl.program_id(1)
        rows = i * bm + jax.lax.broadcasted_iota(jnp.int32, (bm, bn), 0)
        cols = j * bn + jax.lax.broadcasted_iota(jnp.int32, (bm, bn), 1)
        dval = jnp.where(rows < k_real, 1.0, 0.0).astype(jnp.float32)
        res = jnp.where(rows == cols, dval, acc_ref[...])
        o_ref[...] = res
        ot_ref[...] = res.T


def _pool_mm(R, CT, k_real):
    Kp, Np = R.shape
    bm = _pick_block(Kp)
    bn = bm
    bk = _pick_block(Np)
    nk = Np // bk
    grid = (Kp // bm, Kp // bn, nk)
    out, out_t = pl.pallas_call(
        functools.partial(_pool_mm_kernel, nk=nk, bm=bm, bn=bn, k_real=k_real),
        grid=grid,
        in_specs=[
            pl.BlockSpec((bm, bk), lambda i, j, k: (i, k)),
            pl.BlockSpec((bn, bk), lambda i, j, k: (j, k)),
        ],
        out_specs=[
            pl.BlockSpec((bm, bn), lambda i, j, k: (i, j)),
            pl.BlockSpec((bn, bm), lambda i, j, k: (j, i)),
        ],
        out_shape=[
            jax.ShapeDtypeStruct((Kp, Kp), jnp.float32),
            jax.ShapeDtypeStruct((Kp, Kp), jnp.float32),
        ],
        scratch_shapes=[pltpu.VMEM((bm, bn), jnp.float32)],
        compiler_params=pltpu.CompilerParams(
            dimension_semantics=("parallel", "parallel", "arbitrary")),
    )(R, CT)
    return out, out_t


def _agg_mm_kernel(m_ref, dr_ref, dc_ref, fx_ref, z_ref, o_ref, acc_ref,
                   *, nk, bm, bk):
    @pl.when(pl.program_id(1) == 0)
    def _init():
        acc_ref[...] = jnp.zeros_like(acc_ref)

    i = pl.program_id(0)
    k = pl.program_id(1)
    rows = i * bm + jax.lax.broadcasted_iota(jnp.int32, (bm, bk), 0)
    cols = k * bk + jax.lax.broadcasted_iota(jnp.int32, (bm, bk), 1)
    mfix = m_ref[...] + jnp.where(rows == cols, fx_ref[...], 0.0)
    an = (dr_ref[...] * mfix * dc_ref[...]).astype(jnp.bfloat16)
    acc_ref[...] += jnp.dot(an, z_ref[...].astype(jnp.bfloat16),
                            preferred_element_type=jnp.float32)

    @pl.when(pl.program_id(1) == nk - 1)
    def _fin():
        o_ref[...] = acc_ref[...]


def _agg_mm(M, zs, dinv, dfix):
    Np = M.shape[0]
    F = zs.shape[1]
    bm = _pick_block(Np)
    bk = _pick_block(Np)
    nk = Np // bk
    grid = (Np // bm, nk)
    dinv_row = dinv.reshape(1, Np)
    return pl.pallas_call(
        functools.partial(_agg_mm_kernel, nk=nk, bm=bm, bk=bk),
        grid=grid,
        in_specs=[
            pl.BlockSpec((bm, bk), lambda i, k: (i, k)),
            pl.BlockSpec((bm, 1), lambda i, k: (i, 0)),
            pl.BlockSpec((1, bk), lambda i, k: (0, k)),
            pl.BlockSpec((bm, 1), lambda i, k: (i, 0)),
            pl.BlockSpec((bk, F), lambda i, k: (k, 0)),
        ],
        out_specs=pl.BlockSpec((bm, F), lambda i, k: (i, 0)),
        out_shape=jax.ShapeDtypeStruct((Np, F), jnp.float32),
        scratch_shapes=[pltpu.VMEM((bm, F), jnp.float32)],
        compiler_params=pltpu.CompilerParams(
            dimension_semantics=("parallel", "arbitrary")),
    )(M, dinv, dinv_row, dfix, zs)


def _transform_kernel(z_ref, w_ref, o_ref):
    o_ref[...] = jnp.dot(z_ref[...].astype(jnp.bfloat16),
                         w_ref[...].astype(jnp.bfloat16),
                         preferred_element_type=jnp.float32)


def _transform(z, W):
    Np, Din = z.shape
    F = W.shape[1]
    bm = _pick_block(Np)
    grid = (Np // bm,)
    return pl.pallas_call(
        _transform_kernel,
        grid=grid,
        in_specs=[
            pl.BlockSpec((bm, Din), lambda i: (i, 0)),
            pl.BlockSpec((Din, F), lambda i: (0, 0)),
        ],
        out_specs=pl.BlockSpec((bm, F), lambda i: (i, 0)),
        out_shape=jax.ShapeDtypeStruct((Np, F), jnp.float32),
    )(z, W)


def _pad_to(n):
    return max(256, ((n + 255) // 256) * 256)


def kernel(x, edge_index, batch, d0W, d0b, d1W, d1b, d2W, d2b, d3W, d3b,
           d4W, d4b, p0, p1, p2, p3, u0W, u0b, u1W, u1b, u2W, u2b, u3W, u3b,
           c1W, c1b, c2W, c2b, oW, ob):
    n0 = x.shape[0]
    depth = 4
    reals = [n0]
    for _ in range(depth):
        reals.append(int(math.ceil(0.5 * reals[-1])))
    pads = [_pad_to(r) for r in reals]
    P0 = pads[0]

    dW = [d0W, d1W, d2W, d3W, d4W]
    db = [d0b, d1b, d2b, d3b, d4b]
    pw = [p0, p1, p2, p3]
    uW = [u0W, u1W, u2W, u3W]
    ub = [u0b, u1b, u2b, u3b]

    src, dst = edge_index[0], edge_index[1]
    A0 = jnp.zeros((P0, P0), jnp.float32).at[src, dst].add(1.0)
    AT0 = jnp.zeros((P0, P0), jnp.float32).at[dst, src].add(1.0)

    indeg = jnp.zeros((P0,), jnp.float32).at[dst].add(1.0)
    diag0 = jnp.zeros((P0,), jnp.float32).at[src].add(
        jnp.where(src == dst, 1.0, 0.0))
    fix0 = jnp.where(diag0 == 0.0, 2.0, 0.0)
    deg0 = indeg + fix0
    dinv0 = jnp.where(deg0 > 0, 1.0 / jnp.sqrt(jnp.maximum(deg0, 1e-30)), 0.0)
    dinv0_c = dinv0[:, None]

    xp = jnp.zeros((P0, x.shape[1]), jnp.float32).at[:n0, :].set(x)

    def rowmask(Pp, k_real):
        return (jnp.arange(Pp) < k_real)[:, None]

    fix0_c = fix0[:, None]

    zs = _transform(xp, d0W)
    agg = _agg_mm(AT0, zs, dinv0_c, fix0_c)
    h = jax.nn.relu(agg + d0b[None, :])
    h = jnp.where(rowmask(P0, n0), h, 0.0)

    xs = [h]
    dinvs = [dinv0_c]
    dfixs = [fix0_c]
    ats = [AT0]
    perms = []
    Aprev = A0
    ATprev = AT0
    prev_is_level0 = True

    for i in range(1, depth + 1):
        n_real = reals[i - 1]
        k_real = reals[i]
        Pp_prev = pads[i - 1]
        Kp = pads[i]
        w = pw[i - 1]

        score = jnp.tanh((h @ w) / jnp.linalg.norm(w))
        score = jnp.where(jnp.arange(Pp_prev) < n_real, score, -jnp.inf)
        vals, perm = jax.lax.top_k(score, k_real)
        pad_idx = jnp.full((Kp - k_real,), Pp_prev - 1, jnp.int32)
        perm_pad = jnp.concatenate([perm.astype(jnp.int32), pad_idx])
        vals_pad = jnp.concatenate([vals, jnp.zeros((Kp - k_real,), jnp.float32)])

        R = jnp.take(Aprev, perm_pad, axis=0)
        CT = jnp.take(ATprev, perm_pad, axis=0)
        if prev_is_level0:
            ar = jnp.arange(k_real)
            R = R.at[ar, perm].set(1.0)
            CT = CT.at[ar, perm].set(1.0)
        At, AtT = _pool_mm(R.astype(jnp.bfloat16), CT.astype(jnp.bfloat16),
                           k_real)

        h = jnp.take(h, perm_pad, axis=0) * vals_pad[:, None]

        deg = jnp.sum(At, axis=0) + 1.0
        dinv = (1.0 / jnp.sqrt(deg))[:, None]
        ones = jnp.ones((Kp, 1), jnp.float32)
        zs = _transform(h, dW[i])
        agg = _agg_mm(AtT, zs, dinv, ones)
        h = jax.nn.relu(agg + db[i][None, :])
        h = jnp.where(rowmask(Kp, k_real), h, 0.0)

        perms.append((perm_pad, k_real))
        if i < depth:
            xs.append(h)
            dinvs.append(dinv)
            dfixs.append(ones)
            ats.append(AtT)
        Aprev = At
        ATprev = AtT
        prev_is_level0 = False

    for i in range(depth):
        j = depth - 1 - i
        res = xs[j]
        perm_pad, _k = perms[j]
        dinv = dinvs[j]
        up = jnp.zeros_like(res).at[perm_pad].set(h)
        h2 = res + up
        zs = _transform(h2, uW[i])
        agg = _agg_mm(ats[j], zs, dinv, dfixs[j])
        h = agg + ub[i][None, :]
        if i < depth - 1:
            h = jax.nn.relu(h)
            h = jnp.where(rowmask(pads[j], reals[j]), h, 0.0)

    K = 30
    last = jnp.where(jnp.arange(P0) < n0, h[:, -1], -jnp.inf)
    _sv, sidx = jax.lax.top_k(last, K)
    pooled = jnp.take(h, sidx, axis=0)

    y1 = jax.nn.relu(pooled @ c1W[:, 0, :].T + c1b[None, :])
    z = jnp.maximum(y1[0::2, :], y1[1::2, :])
    y2 = c2b[None, :]
    for dt in range(5):
        y2 = y2 + z[dt:dt + 11, :] @ c2W[:, :, dt].T
    y2 = jax.nn.relu(y2)
    vec = y2.T.reshape(1, -1)
    out = jax.nn.relu(vec @ oW + ob[None, :])
    return jax.nn.relu(out)
